# baseline (speedup 1.0000x reference)
import functools

import jax
import jax.numpy as jnp
from jax import lax
from jax.experimental import pallas as pl
from jax.experimental.pallas import tpu as pltpu
from jax.experimental.pallas import tpu_sc as plsc

_NUM_CORES = 2
_NUM_SUBCORES = 16
_NW = _NUM_CORES * _NUM_SUBCORES
_CHUNK = 64


def _sc_bcast_body(table_hbm, out_hbm, buf):
    batch = out_hbm.shape[0]
    seq = table_hbm.shape[0]
    rows_per_w = seq // _NW
    wid = lax.axis_index("s") * _NUM_CORES + lax.axis_index("c")
    base = wid * rows_per_w
    for c in range(rows_per_w // _CHUNK):
        r0 = base + c * _CHUNK
        pltpu.sync_copy(table_hbm.at[pl.ds(r0, _CHUNK)], buf)
        for b in range(batch):
            pltpu.sync_copy(buf, out_hbm.at[b, pl.ds(r0, _CHUNK)])


def _tc_bcast_body(t_ref, o_ref):
    o_ref[...] = t_ref[...][None]


def kernel(x, table):
    batch = x.shape[0]
    seq, dim = table.shape
    bs = 2048
    out = pl.pallas_call(
        _tc_bcast_body,
        grid=(seq // bs, batch),
        in_specs=[pl.BlockSpec((bs, dim), lambda s, b: (s, 0))],
        out_specs=pl.BlockSpec((1, bs, dim), lambda s, b: (b, s, 0)),
        out_shape=jax.ShapeDtypeStruct((batch, seq, dim), table.dtype),
    )(table)
    return out


# speedup vs baseline: 1.0667x; 1.0667x over previous
import functools

import jax
import jax.numpy as jnp
from jax import lax
from jax.experimental import pallas as pl
from jax.experimental.pallas import tpu as pltpu
from jax.experimental.pallas import tpu_sc as plsc

_NUM_CORES = 2
_NUM_SUBCORES = 16
_NW = _NUM_CORES * _NUM_SUBCORES
_CHUNK = 64


def _sc_bcast_body(table_hbm, out_hbm, buf):
    batch = out_hbm.shape[0]
    seq = table_hbm.shape[0]
    rows_per_w = seq // _NW
    wid = lax.axis_index("s") * _NUM_CORES + lax.axis_index("c")
    base = wid * rows_per_w
    for c in range(rows_per_w // _CHUNK):
        r0 = base + c * _CHUNK
        pltpu.sync_copy(table_hbm.at[pl.ds(r0, _CHUNK)], buf)
        for b in range(batch):
            pltpu.sync_copy(buf, out_hbm.at[b, pl.ds(r0, _CHUNK)])


def _tc_dma_body(bs, batch, t_ref, o_hbm, sem):
    s = pl.program_id(0)
    copies = [
        pltpu.make_async_copy(t_ref, o_hbm.at[b, pl.ds(s * bs, bs)], sem)
        for b in range(batch)
    ]
    for c in copies:
        c.start()
    for c in copies:
        c.wait()


def kernel(x, table):
    batch = x.shape[0]
    seq, dim = table.shape
    bs = 1024
    out = pl.pallas_call(
        functools.partial(_tc_dma_body, bs, batch),
        grid=(seq // bs,),
        in_specs=[pl.BlockSpec((bs, dim), lambda s: (s, 0))],
        out_specs=pl.BlockSpec(memory_space=pl.ANY),
        out_shape=jax.ShapeDtypeStruct((batch, seq, dim), table.dtype),
        scratch_shapes=[pltpu.SemaphoreType.DMA],
    )(table)
    return out


# speedup vs baseline: 1.0806x; 1.0131x over previous
import functools

import jax
import jax.numpy as jnp
from jax import lax
from jax.experimental import pallas as pl
from jax.experimental.pallas import tpu as pltpu
from jax.experimental.pallas import tpu_sc as plsc

_NUM_CORES = 2
_NUM_SUBCORES = 16
_NW = _NUM_CORES * _NUM_SUBCORES
_CHUNK = 64


def _sc_bcast_body(table_hbm, out_hbm, buf):
    batch = out_hbm.shape[0]
    seq = table_hbm.shape[0]
    rows_per_w = seq // _NW
    wid = lax.axis_index("s") * _NUM_CORES + lax.axis_index("c")
    base = wid * rows_per_w
    for c in range(rows_per_w // _CHUNK):
        r0 = base + c * _CHUNK
        pltpu.sync_copy(table_hbm.at[pl.ds(r0, _CHUNK)], buf)
        for b in range(batch):
            pltpu.sync_copy(buf, out_hbm.at[b, pl.ds(r0, _CHUNK)])


def _tc_manual_body(bs, batch, nsteps, t_hbm, o_hbm, buf, in_sems, out_sems):
    fetches = [
        pltpu.make_async_copy(
            t_hbm.at[pl.ds(i * bs, bs)], buf.at[i % 2], in_sems.at[i % 2])
        for i in range(nsteps)
    ]
    writes = [
        [pltpu.make_async_copy(
            buf.at[i % 2], o_hbm.at[b, pl.ds(i * bs, bs)], out_sems.at[i % 2])
         for b in range(batch)]
        for i in range(nsteps)
    ]
    fetches[0].start()
    for i in range(nsteps):
        if i + 1 < nsteps:
            if i >= 1:
                for w in writes[i - 1]:
                    w.wait()
            fetches[i + 1].start()
        fetches[i].wait()
        for w in writes[i]:
            w.start()
    if nsteps >= 2:
        for w in writes[nsteps - 2]:
            w.wait()
    for w in writes[nsteps - 1]:
        w.wait()


def kernel(x, table):
    batch = x.shape[0]
    seq, dim = table.shape
    bs = 1024
    nsteps = seq // bs
    out = pl.pallas_call(
        functools.partial(_tc_manual_body, bs, batch, nsteps),
        in_specs=[pl.BlockSpec(memory_space=pl.ANY)],
        out_specs=pl.BlockSpec(memory_space=pl.ANY),
        out_shape=jax.ShapeDtypeStruct((batch, seq, dim), table.dtype),
        scratch_shapes=[
            pltpu.VMEM((2, bs, dim), table.dtype),
            pltpu.SemaphoreType.DMA((2,)),
            pltpu.SemaphoreType.DMA((2,)),
        ],
    )(table)
    return out


# speedup vs baseline: 1.1269x; 1.0428x over previous
import functools

import jax
import jax.numpy as jnp
from jax import lax
from jax.experimental import pallas as pl
from jax.experimental.pallas import tpu as pltpu
from jax.experimental.pallas import tpu_sc as plsc

_NUM_CORES = 2
_NUM_SUBCORES = 16
_NW = _NUM_CORES * _NUM_SUBCORES
_CHUNK = 64


def _sc_bcast_body(table_hbm, out_hbm, buf):
    batch = out_hbm.shape[0]
    seq = table_hbm.shape[0]
    rows_per_w = seq // _NW
    wid = lax.axis_index("s") * _NUM_CORES + lax.axis_index("c")
    base = wid * rows_per_w
    for c in range(rows_per_w // _CHUNK):
        r0 = base + c * _CHUNK
        pltpu.sync_copy(table_hbm.at[pl.ds(r0, _CHUNK)], buf)
        for b in range(batch):
            pltpu.sync_copy(buf, out_hbm.at[b, pl.ds(r0, _CHUNK)])


def _tc_manual_body(bs, batch, nsteps, t_hbm, o_hbm, buf, in_sems, out_sems):
    fetches = [
        pltpu.make_async_copy(
            t_hbm.at[pl.ds(i * bs, bs)], buf.at[i % 2], in_sems.at[i % 2])
        for i in range(nsteps)
    ]
    writes = [
        [pltpu.make_async_copy(
            buf.at[i % 2], o_hbm.at[b, pl.ds(i * bs, bs)], out_sems.at[i % 2])
         for b in range(batch)]
        for i in range(nsteps)
    ]
    fetches[0].start()
    for i in range(nsteps):
        if i + 1 < nsteps:
            if i >= 1:
                for w in writes[i - 1]:
                    w.wait()
            fetches[i + 1].start()
        fetches[i].wait()
        for w in writes[i]:
            w.start()
    if nsteps >= 2:
        for w in writes[nsteps - 2]:
            w.wait()
    for w in writes[nsteps - 1]:
        w.wait()


def _tc_bcast_body(t_ref, o_ref):
    o_ref[...] = jnp.broadcast_to(t_ref[...][None], o_ref.shape)


def kernel(x, table):
    batch = x.shape[0]
    seq, dim = table.shape
    bs = 1024
    out = pl.pallas_call(
        _tc_bcast_body,
        grid=(seq // bs,),
        in_specs=[pl.BlockSpec((bs, dim), lambda s: (s, 0))],
        out_specs=pl.BlockSpec((batch, bs, dim), lambda s: (0, s, 0)),
        out_shape=jax.ShapeDtypeStruct((batch, seq, dim), table.dtype),
    )(table)
    return out
